# baseline (speedup 1.0000x reference)
import functools

import jax
import jax.numpy as jnp
from jax import lax
from jax.experimental import pallas as pl
from jax.experimental.pallas import tpu as pltpu
from jax.experimental.pallas import tpu_sc as plsc

HIDDEN = 16
NUM_WORKERS = 32
K1_ROWS = 320
CHUNK = 3200


def _compact_body(table_hbm, tout_hbm, vmem_in, vmem_out, *, vocab):
    wid = lax.axis_index("s") * 2 + lax.axis_index("c")
    n_blocks = vocab // K1_ROWS
    out_rows = K1_ROWS * HIDDEN // 128

    @pl.loop(wid, n_blocks, step=NUM_WORKERS)
    def _block(c):
        r0 = c * K1_ROWS
        pltpu.sync_copy(table_hbm.at[pl.ds(r0, K1_ROWS), :], vmem_in)

        @pl.loop(0, K1_ROWS, unroll=8)
        def _row(i):
            vmem_out[i // 8, pl.ds((i % 8) * HIDDEN, HIDDEN)] = vmem_in[i, :]

        pltpu.sync_copy(vmem_out, tout_hbm.at[pl.ds(c * out_rows, out_rows)])


def _gather_body(ids_hbm, table_hbm, out_hbm,
                 idx_a, idx_b, rows_a, rows_b, sem_a, sem_b,
                 *, rows_per_worker, n_chunks):
    wid = lax.axis_index("s") * 2 + lax.axis_index("c")
    base = wid * rows_per_worker

    idx = (idx_a, idx_b)
    rows = (rows_a, rows_b)
    sems = (sem_a, sem_b)

    prev = None
    for j in range(n_chunks):
        s = j % 2
        off = base + j * CHUNK
        pltpu.sync_copy(ids_hbm.at[pl.ds(off, CHUNK)], idx[s])
        cp = pltpu.async_copy(table_hbm.at[idx[s]], rows[s], sems[s])
        if prev is not None:
            pcp, ps, poff = prev
            pcp.wait()
            pltpu.sync_copy(rows[ps],
                            out_hbm.at[pl.ds(poff, CHUNK), pl.ds(0, HIDDEN)])
        prev = (cp, s, off)
    pcp, ps, poff = prev
    pcp.wait()
    pltpu.sync_copy(rows[ps],
                    out_hbm.at[pl.ds(poff, CHUNK), pl.ds(0, HIDDEN)])


def kernel(input_ids, attention_mask, emb_weight):
    del attention_mask
    batch, seq = input_ids.shape
    vocab = emb_weight.shape[0]
    total = batch * seq
    rows_per_worker = total // NUM_WORKERS
    n_chunks = rows_per_worker // CHUNK

    flat_ids = input_ids.reshape(total).astype(jnp.int32)

    mesh = plsc.VectorSubcoreMesh(core_axis_name="c", subcore_axis_name="s")

    tcompact = pl.kernel(
        functools.partial(_compact_body, vocab=vocab),
        out_type=jax.ShapeDtypeStruct((vocab * HIDDEN // 128, 128),
                                      jnp.float32),
        mesh=mesh,
        scratch_types=[
            pltpu.VMEM((K1_ROWS, HIDDEN), jnp.float32),
            pltpu.VMEM((K1_ROWS * HIDDEN // 128, 128), jnp.float32),
        ],
    )(emb_weight)
    table_compact = tcompact.reshape(vocab, HIDDEN)

    out2d = pl.kernel(
        functools.partial(_gather_body, rows_per_worker=rows_per_worker,
                          n_chunks=n_chunks),
        out_type=jax.ShapeDtypeStruct((total, 128), jnp.float32),
        mesh=mesh,
        scratch_types=[
            pltpu.VMEM((CHUNK,), jnp.int32),
            pltpu.VMEM((CHUNK,), jnp.int32),
            pltpu.VMEM((CHUNK, HIDDEN), jnp.float32),
            pltpu.VMEM((CHUNK, HIDDEN), jnp.float32),
            pltpu.SemaphoreType.DMA,
            pltpu.SemaphoreType.DMA,
        ],
        compiler_params=pltpu.CompilerParams(use_tc_tiling_on_sc=False),
    )(flat_ids, table_compact)

    return out2d.reshape(batch, seq, 128)[:, :, :HIDDEN]


# speedup vs baseline: 1.1981x; 1.1981x over previous
import functools

import jax
import jax.numpy as jnp
from jax import lax
from jax.experimental import pallas as pl
from jax.experimental.pallas import tpu as pltpu
from jax.experimental.pallas import tpu_sc as plsc

HIDDEN = 16
NUM_WORKERS = 32
CHUNK = 3200


def _gather_body(ids_hbm, table_hbm, out_hbm,
                 idx_a, idx_b, rows_a, rows_b, sem_a, sem_b,
                 *, rows_per_worker, n_chunks):
    wid = lax.axis_index("s") * 2 + lax.axis_index("c")
    base = wid * rows_per_worker

    idx = (idx_a, idx_b)
    rows = (rows_a, rows_b)
    sems = (sem_a, sem_b)

    prev = None
    for j in range(n_chunks):
        s = j % 2
        off = base + j * CHUNK
        pltpu.sync_copy(ids_hbm.at[pl.ds(off, CHUNK)], idx[s])
        cp = pltpu.async_copy(table_hbm.at[idx[s]], rows[s], sems[s])
        if prev is not None:
            pcp, ps, poff = prev
            pcp.wait()
            pltpu.sync_copy(rows[ps],
                            out_hbm.at[pl.ds(poff, CHUNK), pl.ds(0, HIDDEN)])
        prev = (cp, s, off)
    pcp, ps, poff = prev
    pcp.wait()
    pltpu.sync_copy(rows[ps],
                    out_hbm.at[pl.ds(poff, CHUNK), pl.ds(0, HIDDEN)])


def kernel(input_ids, attention_mask, emb_weight):
    del attention_mask
    batch, seq = input_ids.shape
    vocab = emb_weight.shape[0]
    total = batch * seq
    rows_per_worker = total // NUM_WORKERS
    n_chunks = rows_per_worker // CHUNK

    flat_ids = input_ids.reshape(total).astype(jnp.int32)

    table = emb_weight

    mesh = plsc.VectorSubcoreMesh(core_axis_name="c", subcore_axis_name="s")
    out2d = pl.kernel(
        functools.partial(_gather_body, rows_per_worker=rows_per_worker,
                          n_chunks=n_chunks),
        out_type=jax.ShapeDtypeStruct((total, 128), jnp.float32),
        mesh=mesh,
        scratch_types=[
            pltpu.VMEM((CHUNK,), jnp.int32),
            pltpu.VMEM((CHUNK,), jnp.int32),
            pltpu.VMEM((CHUNK, HIDDEN), jnp.float32),
            pltpu.VMEM((CHUNK, HIDDEN), jnp.float32),
            pltpu.SemaphoreType.DMA,
            pltpu.SemaphoreType.DMA,
        ],
        compiler_params=pltpu.CompilerParams(use_tc_tiling_on_sc=False),
    )(flat_ids, table)

    one = jnp.float32(1) - (flat_ids[0] >> 31).astype(jnp.float32)
    return out2d.reshape(batch, seq, 128)[:, :, :HIDDEN] * one


# speedup vs baseline: 1.2579x; 1.0499x over previous
import functools

import jax
import jax.numpy as jnp
from jax import lax
from jax.experimental import pallas as pl
from jax.experimental.pallas import tpu as pltpu
from jax.experimental.pallas import tpu_sc as plsc

HIDDEN = 16
NUM_WORKERS = 32
CHUNK = 1600


NBUF = 3


def _gather_body(ids_hbm, table_hbm, out_hbm,
                 idx_a, idx_b, idx_c, rows_a, rows_b, rows_c,
                 gsem_a, gsem_b, gsem_c, ssem_a, ssem_b, ssem_c,
                 *, rows_per_worker, n_chunks):
    wid = lax.axis_index("s") * 2 + lax.axis_index("c")
    base = wid * rows_per_worker

    idx = (idx_a, idx_b, idx_c)
    rows = (rows_a, rows_b, rows_c)
    gsems = (gsem_a, gsem_b, gsem_c)
    ssems = (ssem_a, ssem_b, ssem_c)

    gather_cp = [None] * NBUF
    store_cp = [None] * NBUF
    prev = None
    for j in range(n_chunks):
        s = j % NBUF
        off = base + j * CHUNK
        if store_cp[s] is not None:
            store_cp[s].wait()
            store_cp[s] = None
        pltpu.sync_copy(ids_hbm.at[pl.ds(off, CHUNK)], idx[s])
        gather_cp[s] = pltpu.async_copy(table_hbm.at[idx[s]], rows[s], gsems[s])
        if prev is not None:
            ps, poff = prev
            gather_cp[ps].wait()
            store_cp[ps] = pltpu.async_copy(
                rows[ps], out_hbm.at[pl.ds(poff, CHUNK), pl.ds(0, HIDDEN)],
                ssems[ps])
        prev = (s, off)
    ps, poff = prev
    gather_cp[ps].wait()
    store_cp[ps] = pltpu.async_copy(
        rows[ps], out_hbm.at[pl.ds(poff, CHUNK), pl.ds(0, HIDDEN)], ssems[ps])
    for s in range(NBUF):
        if store_cp[s] is not None:
            store_cp[s].wait()


def kernel(input_ids, attention_mask, emb_weight):
    del attention_mask
    batch, seq = input_ids.shape
    vocab = emb_weight.shape[0]
    total = batch * seq
    rows_per_worker = total // NUM_WORKERS
    n_chunks = rows_per_worker // CHUNK

    flat_ids = input_ids.reshape(total).astype(jnp.int32)

    table = emb_weight

    mesh = plsc.VectorSubcoreMesh(core_axis_name="c", subcore_axis_name="s")
    out2d = pl.kernel(
        functools.partial(_gather_body, rows_per_worker=rows_per_worker,
                          n_chunks=n_chunks),
        out_type=jax.ShapeDtypeStruct((total, 128), jnp.float32),
        mesh=mesh,
        scratch_types=(
            [pltpu.VMEM((CHUNK,), jnp.int32)] * NBUF
            + [pltpu.VMEM((CHUNK, HIDDEN), jnp.float32)] * NBUF
            + [pltpu.SemaphoreType.DMA] * (2 * NBUF)
        ),
        compiler_params=pltpu.CompilerParams(use_tc_tiling_on_sc=False),
    )(flat_ids, table)

    return out2d.reshape(batch, seq, 128)[:, :, :HIDDEN]
